# baseline (speedup 1.0000x reference)
import functools

import jax
import jax.numpy as jnp
from jax import lax
from jax.experimental import pallas as pl
from jax.experimental.pallas import tpu as pltpu
from jax.experimental.pallas import tpu_sc as plsc

N = 10000
E = 160000
D_IN = 128
D_E = 16
D = 32
D_EH = 128
STEPS = 6

NC = 2
NS = 16
NW = NC * NS
CHUNK = 128
CPW = 40
EPW = CHUNK * CPW
E_PAD = NW * EPW
STRIPE = 632
N_ACC = NS * STRIPE


GW = 128


@functools.lru_cache(maxsize=None)
def _build_sc_gather():
    mesh = plsc.VectorSubcoreMesh(core_axis_name="c", subcore_axis_name="s")

    @functools.partial(
        pl.kernel, mesh=mesh,
        out_type=jax.ShapeDtypeStruct((E_PAD, GW), jnp.float32),
        scratch_types=[
            pltpu.VMEM((CHUNK,), jnp.int32),
            pltpu.VMEM((CHUNK, GW), jnp.float32),
            pltpu.SemaphoreType.DMA,
        ],
    )
    def k(h_hbm, idx_hbm, out_hbm, idx_v, rows_v, sem):
        wid = lax.axis_index("s") * NC + lax.axis_index("c")
        base = wid * CPW

        def body(j, carry):
            r = base + j
            pltpu.sync_copy(idx_hbm.at[r], idx_v)
            pltpu.async_copy(h_hbm.at[idx_v], rows_v, sem).wait()
            pltpu.sync_copy(rows_v, out_hbm.at[pl.ds(r * CHUNK, CHUNK)])
            return carry

        lax.fori_loop(0, CPW, body, 0)

    return k


def _sc_gather(h, idx2d):
    return _build_sc_gather()(h, idx2d)


@functools.lru_cache(maxsize=None)
def _build_sc_scatter():
    mesh = plsc.VectorSubcoreMesh(core_axis_name="c", subcore_axis_name="s")

    @functools.partial(
        pl.kernel, mesh=mesh,
        out_type=jax.ShapeDtypeStruct((NC, N_ACC, GW), jnp.float32),
        scratch_types=[
            pltpu.VMEM((CHUNK,), jnp.int32),
            pltpu.VMEM((CHUNK, GW), jnp.float32),
            pltpu.VMEM_SHARED((N_ACC, GW), jnp.float32),
            pltpu.SemaphoreType.DMA,
        ],
    )
    def k(msg_hbm, dst_hbm, z_hbm, out_hbm, dst_v, rows_v, acc_sh, sem):
        c = lax.axis_index("c")
        s = lax.axis_index("s")
        wid = s * NC + c
        pltpu.sync_copy(z_hbm, acc_sh.at[pl.ds(s * STRIPE, STRIPE)])
        plsc.subcore_barrier()
        base = wid * CPW

        def body(j, carry):
            r = base + j
            pltpu.sync_copy(dst_hbm.at[r], dst_v)
            pltpu.sync_copy(msg_hbm.at[pl.ds(r * CHUNK, CHUNK)], rows_v)
            pltpu.sync_copy(rows_v, acc_sh.at[dst_v], add=True)
            return carry

        lax.fori_loop(0, CPW, body, 0)
        plsc.subcore_barrier()
        pltpu.sync_copy(acc_sh.at[pl.ds(s * STRIPE, STRIPE)],
                        out_hbm.at[c, pl.ds(s * STRIPE, STRIPE)])

    return k


def _sc_scatter(msg, dst2d, zstripe):
    return _build_sc_scatter()(msg, dst2d, zstripe)


def _lin0_body(x_ref, w_ref, b_ref, o_ref):
    o_ref[:, :D] = jnp.maximum(
        jnp.dot(x_ref[...], w_ref[...], preferred_element_type=jnp.float32)
        + b_ref[...], 0.0)
    o_ref[:, D:] = jnp.zeros((o_ref.shape[0], GW - D), jnp.float32)


def _lin0(n_feat, w, b):
    BN = 2000
    return pl.pallas_call(
        _lin0_body,
        grid=(N // BN,),
        in_specs=[
            pl.BlockSpec((BN, D_IN), lambda i: (i, 0)),
            pl.BlockSpec((D_IN, D), lambda i: (0, 0)),
            pl.BlockSpec((1, D), lambda i: (0, 0)),
        ],
        out_specs=pl.BlockSpec((BN, GW), lambda i: (i, 0)),
        out_shape=jax.ShapeDtypeStruct((N, GW), jnp.float32),
    )(n_feat, w, b)


def _edgenet_body(e_ref, w1_ref, b1_ref, w2_ref, b2_ref, o_ref):
    g = jnp.maximum(
        jnp.dot(e_ref[...], w1_ref[...], preferred_element_type=jnp.float32)
        + b1_ref[...], 0.0)
    o_ref[...] = (
        jnp.dot(g, w2_ref[...], preferred_element_type=jnp.float32)
        + b2_ref[...])


def _edgenet(e_feat_p, w1, b1, w2, b2):
    BE = 1024
    return pl.pallas_call(
        _edgenet_body,
        grid=(E_PAD // BE,),
        in_specs=[
            pl.BlockSpec((BE, D_E), lambda i: (i, 0)),
            pl.BlockSpec((D_E, D_EH), lambda i: (0, 0)),
            pl.BlockSpec((1, D_EH), lambda i: (0, 0)),
            pl.BlockSpec((D_EH, D * D), lambda i: (0, 0)),
            pl.BlockSpec((1, D * D), lambda i: (0, 0)),
        ],
        out_specs=pl.BlockSpec((BE, D * D), lambda i: (i, 0)),
        out_shape=jax.ShapeDtypeStruct((E_PAD, D * D), jnp.float32),
    )(e_feat_p, w1, b1, w2, b2)


_MSG_BE = 512


def _msg_body(hs_ref, we_ref, o_ref):
    hs = hs_ref[:, :D]
    acc = hs[:, 0:1] * we_ref[:, 0:D]
    for dd in range(1, D):
        acc = acc + hs[:, dd:dd + 1] * we_ref[:, dd * D:(dd + 1) * D]
    o_ref[:, :D] = acc
    o_ref[:, D:] = jnp.zeros((_MSG_BE, GW - D), jnp.float32)


def _msg(h_src, we):
    BE = _MSG_BE
    return pl.pallas_call(
        _msg_body,
        grid=(E_PAD // BE,),
        in_specs=[
            pl.BlockSpec((BE, GW), lambda i: (i, 0)),
            pl.BlockSpec((BE, D * D), lambda i: (i, 0)),
        ],
        out_specs=pl.BlockSpec((BE, GW), lambda i: (i, 0)),
        out_shape=jax.ShapeDtypeStruct((E_PAD, GW), jnp.float32),
    )(h_src, we)


def _gru_body(p_ref, h_ref, wih_ref, whh_ref, bih_ref, bhh_ref, cb_ref, o_ref):
    h = h_ref[:, :D]
    agg = p_ref[0][:, :D] + p_ref[1][:, :D] + cb_ref[...]
    m = jnp.maximum(agg, 0.0)
    gi = jnp.dot(m, wih_ref[...], preferred_element_type=jnp.float32) + bih_ref[...]
    gh = jnp.dot(h, whh_ref[...], preferred_element_type=jnp.float32) + bhh_ref[...]
    r = jax.nn.sigmoid(gi[:, 0:D] + gh[:, 0:D])
    z = jax.nn.sigmoid(gi[:, D:2 * D] + gh[:, D:2 * D])
    n = jnp.tanh(gi[:, 2 * D:3 * D] + r * gh[:, 2 * D:3 * D])
    o_ref[:, :D] = (1.0 - z) * n + z * h
    o_ref[:, D:] = jnp.zeros((o_ref.shape[0], GW - D), jnp.float32)


def _gru(partials, h, wih_t, whh_t, bih, bhh, cb):
    BN = 2000
    return pl.pallas_call(
        _gru_body,
        grid=(N // BN,),
        in_specs=[
            pl.BlockSpec((NC, BN, GW), lambda i: (0, i, 0)),
            pl.BlockSpec((BN, GW), lambda i: (i, 0)),
            pl.BlockSpec((D, 3 * D), lambda i: (0, 0)),
            pl.BlockSpec((D, 3 * D), lambda i: (0, 0)),
            pl.BlockSpec((1, 3 * D), lambda i: (0, 0)),
            pl.BlockSpec((1, 3 * D), lambda i: (0, 0)),
            pl.BlockSpec((1, D), lambda i: (0, 0)),
        ],
        out_specs=pl.BlockSpec((BN, GW), lambda i: (i, 0)),
        out_shape=jax.ShapeDtypeStruct((N, GW), jnp.float32),
    )(partials, h, wih_t, whh_t, bih, bhh, cb)


def kernel(n_feat, edge_index, e_feat, lin0_W, lin0_b, en_W1, en_b1, en_W2,
           en_b2, conv_bias, gru_Wih, gru_Whh, gru_bih, gru_bhh):
    pad = E_PAD - E
    src2d = jnp.concatenate(
        [edge_index[0], jnp.zeros((pad,), jnp.int32)]).reshape(NW * CPW, CHUNK)
    dst2d = jnp.concatenate(
        [edge_index[1], jnp.full((pad,), N, jnp.int32)]).reshape(NW * CPW, CHUNK)
    e_feat_p = jnp.concatenate(
        [e_feat, jnp.zeros((pad, D_E), jnp.float32)], axis=0)
    zstripe = jnp.zeros((STRIPE, GW), jnp.float32)

    h = _lin0(n_feat, lin0_W, lin0_b.reshape(1, D))
    we = _edgenet(e_feat_p, en_W1, en_b1.reshape(1, D_EH), en_W2,
                  en_b2.reshape(1, D * D))

    wih_t = gru_Wih.T
    whh_t = gru_Whh.T
    bih = gru_bih.reshape(1, 3 * D)
    bhh = gru_bhh.reshape(1, 3 * D)
    cb = conv_bias.reshape(1, D)

    for _ in range(STEPS):
        h_src = _sc_gather(h, src2d)
        msg = _msg(h_src, we)
        partials = _sc_scatter(msg, dst2d, zstripe)
        h = _gru(partials, h, wih_t, whh_t, bih, bhh, cb)
    return h[:, :D]
